# Initial kernel scaffold; baseline (speedup 1.0000x reference)
import functools

import jax
import jax.numpy as jnp
from jax import lax
from jax.experimental import pallas as pl
from jax.experimental.pallas import tpu as pltpu
from jax.experimental.pallas import tpu_sc as plsc

N_NODES = 10000
N_EDGES = 320000
FEATS = 128

NC = 2
NS = 16
NW = NC * NS
EPW = N_EDGES // NW
B = 80
NBLK = EPW // B
NPS = N_NODES // NS

_mesh = plsc.VectorSubcoreMesh(core_axis_name="c", subcore_axis_name="s")


@functools.partial(
    pl.kernel,
    out_type=jax.ShapeDtypeStruct((NC, N_NODES, 16), jnp.float32),
    mesh=_mesh,
    scratch_types=[
        pltpu.VMEM((B,), jnp.int32),
        pltpu.VMEM((B, 16), jnp.float32),
        pltpu.VMEM_SHARED((N_NODES, 16), jnp.float32),
    ],
)
def _sc_degree(dst_hbm, zeros_hbm, ones_hbm, out_hbm, idx_v, ones_v, acc):
    c = lax.axis_index("c")
    s = lax.axis_index("s")
    wid = s * NC + c
    pltpu.sync_copy(ones_hbm, ones_v)
    pltpu.sync_copy(zeros_hbm.at[pl.ds(s * NPS, NPS)], acc.at[pl.ds(s * NPS, NPS)])
    plsc.subcore_barrier()

    def blk(i, carry):
        base = wid * EPW + i * B
        pltpu.sync_copy(dst_hbm.at[pl.ds(base, B)], idx_v)
        pltpu.sync_copy(ones_v, acc.at[idx_v], add=True)
        return carry

    lax.fori_loop(0, NBLK, blk, 0)
    plsc.subcore_barrier()
    pltpu.sync_copy(acc.at[pl.ds(s * NPS, NPS)], out_hbm.at[c, pl.ds(s * NPS, NPS)])


@functools.partial(
    pl.kernel,
    out_type=jax.ShapeDtypeStruct((NC, N_NODES, FEATS), jnp.float32),
    mesh=_mesh,
    scratch_types=[
        pltpu.VMEM((B,), jnp.int32),
        pltpu.VMEM((B,), jnp.int32),
        pltpu.VMEM((N_NODES,), jnp.float32),
        pltpu.VMEM((B, FEATS), jnp.float32),
        pltpu.VMEM((B, FEATS), jnp.float32),
        pltpu.VMEM((B,), jnp.float32),
        pltpu.SemaphoreType.DMA,
        pltpu.VMEM_SHARED((N_NODES, FEATS), jnp.float32),
    ],
)
def _sc_edges(src_hbm, dst_hbm, x_hbm, w_hbm, dinv_hbm, hinit_hbm, out_hbm,
              src_v, dst_v, dinv_v, xb, wb, norm_v, sem, acc):
    c = lax.axis_index("c")
    s = lax.axis_index("s")
    wid = s * NC + c
    pltpu.sync_copy(dinv_hbm, dinv_v)
    pltpu.sync_copy(hinit_hbm.at[pl.ds(s * NPS, NPS)], acc.at[pl.ds(s * NPS, NPS)])
    plsc.subcore_barrier()

    def blk(i, carry):
        base = wid * EPW + i * B
        pltpu.sync_copy(src_hbm.at[pl.ds(base, B)], src_v)
        pltpu.sync_copy(dst_hbm.at[pl.ds(base, B)], dst_v)
        gather = pltpu.async_copy(x_hbm.at[src_v], xb, sem)
        pltpu.sync_copy(w_hbm.at[pl.ds(base, B)], wb)
        gather.wait()
        for k in range(B // 16):
            sl = pl.ds(k * 16, 16)
            si = src_v[sl]
            di = dst_v[sl]
            norm_v[sl] = plsc.load_gather(dinv_v, [si]) * plsc.load_gather(dinv_v, [di])

        def edge(e, ecarry):
            nv = norm_v[e]
            for j in range(FEATS // 16):
                fsl = pl.ds(j * 16, 16)
                xb[e, fsl] = jnp.maximum(xb[e, fsl] + wb[e, fsl], 0.0) * nv
            return ecarry

        lax.fori_loop(0, B, edge, 0)
        pltpu.sync_copy(xb, acc.at[dst_v], add=True)
        return carry

    lax.fori_loop(0, NBLK, blk, 0)
    plsc.subcore_barrier()
    pltpu.sync_copy(acc.at[pl.ds(s * NPS, NPS)], out_hbm.at[c, pl.ds(s * NPS, NPS)])


def _tc_node_body(feat_ref, w_ref, root_ref, p0_ref, p1_ref,
                  x_ref, dinv_ref, hinit_ref):
    x = jnp.dot(feat_ref[...], w_ref[...], preferred_element_type=jnp.float32)
    deg = 1.0 + p0_ref[0, :, 0:1] + p1_ref[0, :, 0:1]
    x_ref[...] = x
    dinv_ref[...] = lax.rsqrt(deg)
    hinit_ref[...] = (0.5 * jnp.maximum(x + root_ref[...], 0.0)) * (1.0 / deg)


def _tc_w_body(ef_ref, we_ref, be_ref, w_ref):
    w_ref[...] = (
        jnp.dot(ef_ref[...], we_ref[...], preferred_element_type=jnp.float32)
        + be_ref[...]
    )


def _tc_merge_body(a_ref, b_ref, o_ref):
    o_ref[...] = a_ref[0] + b_ref[0]


def kernel(feat, edge_index, edge_feat, W, We, be, root_emb):
    src = edge_index[0].astype(jnp.int32)
    dst = edge_index[1].astype(jnp.int32)

    zeros16 = jnp.zeros((N_NODES, 16), jnp.float32)
    ones16 = jnp.ones((B, 16), jnp.float32)
    deg_parts = _sc_degree(dst, zeros16, ones16)

    rn = 1000
    x, dinv, hinit_half = pl.pallas_call(
        _tc_node_body,
        grid=(N_NODES // rn,),
        in_specs=[
            pl.BlockSpec((rn, FEATS), lambda i: (i, 0)),
            pl.BlockSpec((FEATS, FEATS), lambda i: (0, 0)),
            pl.BlockSpec((1, FEATS), lambda i: (0, 0)),
            pl.BlockSpec((1, rn, 16), lambda i: (0, i, 0)),
            pl.BlockSpec((1, rn, 16), lambda i: (1, i, 0)),
        ],
        out_specs=[
            pl.BlockSpec((rn, FEATS), lambda i: (i, 0)),
            pl.BlockSpec((rn, 1), lambda i: (i, 0)),
            pl.BlockSpec((rn, FEATS), lambda i: (i, 0)),
        ],
        out_shape=[
            jax.ShapeDtypeStruct((N_NODES, FEATS), jnp.float32),
            jax.ShapeDtypeStruct((N_NODES, 1), jnp.float32),
            jax.ShapeDtypeStruct((N_NODES, FEATS), jnp.float32),
        ],
    )(feat, W, root_emb, deg_parts, deg_parts)
    dinv = dinv.reshape((N_NODES,))

    ef_pad = jnp.pad(edge_feat, ((0, 0), (0, 1)))
    we_pad = jnp.pad(We, ((0, 1), (0, 0)))
    re = 4000
    w = pl.pallas_call(
        _tc_w_body,
        grid=(N_EDGES // re,),
        in_specs=[
            pl.BlockSpec((re, 8), lambda i: (i, 0)),
            pl.BlockSpec((8, FEATS), lambda i: (0, 0)),
            pl.BlockSpec((1, FEATS), lambda i: (0, 0)),
        ],
        out_specs=pl.BlockSpec((re, FEATS), lambda i: (i, 0)),
        out_shape=jax.ShapeDtypeStruct((N_EDGES, FEATS), jnp.float32),
    )(ef_pad, we_pad, be.reshape(1, FEATS))

    parts = _sc_edges(src, dst, x, w, dinv, hinit_half)

    out = pl.pallas_call(
        _tc_merge_body,
        grid=(N_NODES // rn,),
        in_specs=[
            pl.BlockSpec((1, rn, FEATS), lambda i: (0, i, 0)),
            pl.BlockSpec((1, rn, FEATS), lambda i: (1, i, 0)),
        ],
        out_specs=pl.BlockSpec((rn, FEATS), lambda i: (i, 0)),
        out_shape=jax.ShapeDtypeStruct((N_NODES, FEATS), jnp.float32),
    )(parts, parts)
    return out


# speedup vs baseline: 8.4796x; 8.4796x over previous
import functools

import jax
import jax.numpy as jnp
from jax import lax
from jax.experimental import pallas as pl
from jax.experimental.pallas import tpu as pltpu
from jax.experimental.pallas import tpu_sc as plsc

N_NODES = 10000
N_EDGES = 320000
FEATS = 128
NP = 10240

NC = 2
NS = 16
NW = NC * NS
EPW = N_EDGES // NW
B = 80
NBLK = EPW // B
NPS = NP // NS

_mesh = plsc.VectorSubcoreMesh(core_axis_name="c", subcore_axis_name="s")


@functools.partial(
    pl.kernel,
    out_type=jax.ShapeDtypeStruct((NC, NP, 16), jnp.float32),
    mesh=_mesh,
    scratch_types=[
        pltpu.VMEM((B,), jnp.int32),
        pltpu.VMEM((B, 16), jnp.float32),
        pltpu.VMEM_SHARED((NP, 16), jnp.float32),
    ],
)
def _sc_degree(dst_hbm, zeros_hbm, ones_hbm, out_hbm, idx_v, ones_v, acc):
    c = lax.axis_index("c")
    s = lax.axis_index("s")
    wid = s * NC + c
    pltpu.sync_copy(ones_hbm, ones_v)
    pltpu.sync_copy(zeros_hbm.at[pl.ds(s * NPS, NPS)], acc.at[pl.ds(s * NPS, NPS)])
    plsc.subcore_barrier()

    def blk(i, carry):
        base = wid * EPW + i * B
        pltpu.sync_copy(dst_hbm.at[pl.ds(base, B)], idx_v)
        pltpu.sync_copy(ones_v, acc.at[idx_v], add=True)
        return carry

    lax.fori_loop(0, NBLK, blk, 0)
    plsc.subcore_barrier()
    pltpu.sync_copy(acc.at[pl.ds(s * NPS, NPS)], out_hbm.at[c, pl.ds(s * NPS, NPS)])


@functools.partial(
    pl.kernel,
    out_type=jax.ShapeDtypeStruct((NC, NP, FEATS), jnp.float32),
    mesh=_mesh,
    scratch_types=[
        pltpu.VMEM((B,), jnp.int32),
        pltpu.VMEM((B,), jnp.int32),
        pltpu.VMEM((B, FEATS), jnp.float32),
        pltpu.VMEM((B,), jnp.float32),
        pltpu.VMEM((B,), jnp.float32),
        pltpu.SemaphoreType.DMA,
        pltpu.SemaphoreType.DMA,
        pltpu.VMEM_SHARED((NP, FEATS), jnp.float32),
    ],
)
def _sc_edges(src_hbm, dst_hbm, x_hbm, w_hbm, dinv_hbm, hinit_hbm, out_hbm,
              src_v, dst_v, xb, ns_v, nd_v, sem_x, sem_n, acc):
    c = lax.axis_index("c")
    s = lax.axis_index("s")
    wid = s * NC + c
    pltpu.sync_copy(hinit_hbm.at[pl.ds(s * NPS, NPS)], acc.at[pl.ds(s * NPS, NPS)])
    plsc.subcore_barrier()

    def blk(i, carry):
        base = wid * EPW + i * B
        pltpu.sync_copy(src_hbm.at[pl.ds(base, B)], src_v)
        pltpu.sync_copy(dst_hbm.at[pl.ds(base, B)], dst_v)
        g_ns = pltpu.async_copy(dinv_hbm.at[src_v], ns_v, sem_n)
        g_nd = pltpu.async_copy(dinv_hbm.at[dst_v], nd_v, sem_n)
        pltpu.sync_copy(w_hbm.at[pl.ds(base, B)], xb)
        g_x = pltpu.async_copy(x_hbm.at[src_v], xb, sem_x, add=True)
        g_ns.wait()
        g_nd.wait()
        g_x.wait()

        def grp(g, gcarry):
            sl = pl.ds(g * 16, 16)
            nrm = ns_v[sl] * nd_v[sl]
            for l in range(16):
                nv = nrm[l]
                e = g * 16 + l
                for j in range(FEATS // 16):
                    fsl = pl.ds(j * 16, 16)
                    xb[e, fsl] = jnp.maximum(xb[e, fsl], 0.0) * nv
            return gcarry

        lax.fori_loop(0, B // 16, grp, 0)
        pltpu.sync_copy(xb, acc.at[dst_v], add=True)
        return carry

    lax.fori_loop(0, NBLK, blk, 0)
    plsc.subcore_barrier()
    pltpu.sync_copy(acc.at[pl.ds(s * NPS, NPS)], out_hbm.at[c, pl.ds(s * NPS, NPS)])


def _tc_node_body(feat_ref, w_ref, root_ref, p0_ref, p1_ref,
                  x_ref, dinv_ref, hinit_ref):
    x = jnp.dot(feat_ref[...], w_ref[...], preferred_element_type=jnp.float32)
    deg = 1.0 + p0_ref[0, :, 0:1] + p1_ref[0, :, 0:1]
    x_ref[...] = x
    dinv_ref[...] = lax.rsqrt(deg)
    hinit_ref[...] = (0.5 * jnp.maximum(x + root_ref[...], 0.0)) * (1.0 / deg)


def _tc_w_body(ef_ref, we_ref, be_ref, w_ref):
    w_ref[...] = (
        jnp.dot(ef_ref[...], we_ref[...], preferred_element_type=jnp.float32)
        + be_ref[...]
    )


def _tc_merge_body(a_ref, b_ref, o_ref):
    o_ref[...] = a_ref[0] + b_ref[0]


def kernel(feat, edge_index, edge_feat, W, We, be, root_emb):
    src = edge_index[0].astype(jnp.int32)
    dst = edge_index[1].astype(jnp.int32)

    feat_p = jnp.pad(feat, ((0, NP - N_NODES), (0, 0)))
    zeros16 = jnp.zeros((NP, 16), jnp.float32)
    ones16 = jnp.ones((B, 16), jnp.float32)
    deg_parts = _sc_degree(dst, zeros16, ones16)

    rn = 640
    x, dinv, hinit_half = pl.pallas_call(
        _tc_node_body,
        grid=(NP // rn,),
        in_specs=[
            pl.BlockSpec((rn, FEATS), lambda i: (i, 0)),
            pl.BlockSpec((FEATS, FEATS), lambda i: (0, 0)),
            pl.BlockSpec((1, FEATS), lambda i: (0, 0)),
            pl.BlockSpec((1, rn, 16), lambda i: (0, i, 0)),
            pl.BlockSpec((1, rn, 16), lambda i: (1, i, 0)),
        ],
        out_specs=[
            pl.BlockSpec((rn, FEATS), lambda i: (i, 0)),
            pl.BlockSpec((rn, 1), lambda i: (i, 0)),
            pl.BlockSpec((rn, FEATS), lambda i: (i, 0)),
        ],
        out_shape=[
            jax.ShapeDtypeStruct((NP, FEATS), jnp.float32),
            jax.ShapeDtypeStruct((NP, 1), jnp.float32),
            jax.ShapeDtypeStruct((NP, FEATS), jnp.float32),
        ],
    )(feat_p, W, root_emb, deg_parts, deg_parts)
    dinv = dinv.reshape((NP,))

    ef_pad = jnp.pad(edge_feat, ((0, 0), (0, 1)))
    we_pad = jnp.pad(We, ((0, 1), (0, 0)))
    re = 4000
    w = pl.pallas_call(
        _tc_w_body,
        grid=(N_EDGES // re,),
        in_specs=[
            pl.BlockSpec((re, 8), lambda i: (i, 0)),
            pl.BlockSpec((8, FEATS), lambda i: (0, 0)),
            pl.BlockSpec((1, FEATS), lambda i: (0, 0)),
        ],
        out_specs=pl.BlockSpec((re, FEATS), lambda i: (i, 0)),
        out_shape=jax.ShapeDtypeStruct((N_EDGES, FEATS), jnp.float32),
    )(ef_pad, we_pad, be.reshape(1, FEATS))

    parts = _sc_edges(src, dst, x, w, dinv, hinit_half)

    out = pl.pallas_call(
        _tc_merge_body,
        grid=(NP // rn,),
        in_specs=[
            pl.BlockSpec((1, rn, FEATS), lambda i: (0, i, 0)),
            pl.BlockSpec((1, rn, FEATS), lambda i: (1, i, 0)),
        ],
        out_specs=pl.BlockSpec((rn, FEATS), lambda i: (i, 0)),
        out_shape=jax.ShapeDtypeStruct((NP, FEATS), jnp.float32),
    )(parts, parts)
    return out[:N_NODES]


# speedup vs baseline: 11.2206x; 1.3233x over previous
import functools

import jax
import jax.numpy as jnp
from jax import lax
from jax.experimental import pallas as pl
from jax.experimental.pallas import tpu as pltpu
from jax.experimental.pallas import tpu_sc as plsc

N_NODES = 10000
N_EDGES = 320000
FEATS = 128
NP = 10240

NC = 2
NS = 16
NW = NC * NS
EPW = N_EDGES // NW
B = 80
NBLK = EPW // B
NPS = NP // NS

_mesh = plsc.VectorSubcoreMesh(core_axis_name="c", subcore_axis_name="s")


@functools.partial(
    pl.kernel,
    out_type=jax.ShapeDtypeStruct((NC, NP, 16), jnp.float32),
    mesh=_mesh,
    scratch_types=[
        pltpu.VMEM((NBLK, B), jnp.int32),
        pltpu.VMEM((B, 16), jnp.float32),
        pltpu.SemaphoreType.DMA,
        pltpu.SemaphoreType.DMA,
        pltpu.SemaphoreType.DMA,
        pltpu.SemaphoreType.DMA,
        pltpu.SemaphoreType.DMA,
        pltpu.VMEM_SHARED((NP, 16), jnp.float32),
    ],
)
def _sc_degree(dst_hbm, zeros_hbm, ones_hbm, out_hbm,
               id_all, ones_v, s0, s1, s2, s3, s4, acc):
    c = lax.axis_index("c")
    s = lax.axis_index("s")
    wid = s * NC + c
    sems = (s0, s1, s2, s3, s4)
    pltpu.sync_copy(ones_hbm, ones_v)
    pltpu.sync_copy(dst_hbm.at[wid], id_all)
    pltpu.sync_copy(zeros_hbm.at[pl.ds(s * NPS, NPS)], acc.at[pl.ds(s * NPS, NPS)])
    plsc.subcore_barrier()

    def q_body(q, carry):
        for j in range(5):
            r = q * 5 + j

            @pl.when(q > 0)
            def _():
                pltpu.make_async_copy(ones_v, acc.at[id_all.at[0]], sems[j]).wait()

            pltpu.async_copy(ones_v, acc.at[id_all.at[r]], sems[j], add=True)
        return carry

    lax.fori_loop(0, NBLK // 5, q_body, 0)
    for j in range(5):
        pltpu.make_async_copy(ones_v, acc.at[id_all.at[0]], sems[j]).wait()
    plsc.subcore_barrier()
    pltpu.sync_copy(acc.at[pl.ds(s * NPS, NPS)], out_hbm.at[c, pl.ds(s * NPS, NPS)])


@functools.partial(
    pl.kernel,
    out_type=jax.ShapeDtypeStruct((NC, NP, FEATS), jnp.float32),
    mesh=_mesh,
    scratch_types=[
        pltpu.VMEM((EPW,), jnp.int32),
        pltpu.VMEM((EPW,), jnp.int32),
        pltpu.VMEM((B, FEATS), jnp.float32),
        pltpu.VMEM((B, FEATS), jnp.float32),
        pltpu.VMEM((2, B), jnp.float32),
        pltpu.VMEM((2, B), jnp.float32),
        pltpu.VMEM((B,), jnp.int32),
        pltpu.VMEM((B,), jnp.int32),
        pltpu.SemaphoreType.DMA,
        pltpu.SemaphoreType.DMA,
        pltpu.SemaphoreType.DMA,
        pltpu.SemaphoreType.DMA,
        pltpu.SemaphoreType.DMA,
        pltpu.SemaphoreType.DMA,
        pltpu.SemaphoreType.DMA,
        pltpu.SemaphoreType.DMA,
        pltpu.VMEM_SHARED((NP, FEATS), jnp.float32),
    ],
)
def _sc_edges(src_hbm, dst_hbm, x_hbm, w_hbm, dinv_hbm, hinit_hbm, out_hbm,
              srcall, dstall, xb0, xb1, nn0, nn1, dv0, dv1,
              sn0, sn1, sw0, sw1, sx0, sx1, ss0, ss1, acc):
    c = lax.axis_index("c")
    s = lax.axis_index("s")
    wid = s * NC + c
    ebase = wid * EPW

    pltpu.sync_copy(src_hbm.at[pl.ds(ebase, EPW)], srcall)
    pltpu.sync_copy(dst_hbm.at[pl.ds(ebase, EPW)], dstall)
    pltpu.sync_copy(hinit_hbm.at[pl.ds(s * NPS, NPS)], acc.at[pl.ds(s * NPS, NPS)])

    bufs = ((xb0, nn0, dv0, sn0, sw0, sx0, ss0),
            (xb1, nn1, dv1, sn1, sw1, sx1, ss1))

    def w_issue(i, xb, sem):
        pltpu.async_copy(w_hbm.at[pl.ds(ebase + i * B, B)], xb, sem)

    def w_wait(xb, sem):
        pltpu.make_async_copy(w_hbm.at[pl.ds(0, B)], xb, sem).wait()

    def nn_issue(i, nn, sem):
        pltpu.async_copy(dinv_hbm.at[srcall.at[pl.ds(i * B, B)]], nn.at[0], sem)
        pltpu.async_copy(dinv_hbm.at[dstall.at[pl.ds(i * B, B)]], nn.at[1], sem)

    def nn_wait(nn, sem):
        pltpu.make_async_copy(dinv_hbm.at[pl.ds(0, B)], nn.at[0], sem).wait()
        pltpu.make_async_copy(dinv_hbm.at[pl.ds(0, B)], nn.at[1], sem).wait()

    def x_issue(i, xb, sem):
        pltpu.async_copy(x_hbm.at[srcall.at[pl.ds(i * B, B)]], xb, sem, add=True)

    def x_wait(xb, sem):
        pltpu.make_async_copy(x_hbm.at[pl.ds(0, B)], xb, sem).wait()

    def sc_issue(dv, xb, sem):
        pltpu.async_copy(xb, acc.at[dv], sem, add=True)

    def sc_wait(dv, xb, sem):
        pltpu.make_async_copy(xb, acc.at[dv], sem).wait()

    def compute(i, xb, nn, dv):
        for g5 in range(B // 16):
            sl = pl.ds(g5 * 16, 16)
            dv[sl] = dstall[pl.ds(i * B + g5 * 16, 16)]

        def grp(g, gc):
            sl = pl.ds(g * 16, 16)
            nrm = nn[0, sl] * nn[1, sl]
            for l in range(16):
                nv = nrm[l]
                e = g * 16 + l
                for j in range(FEATS // 16):
                    fsl = pl.ds(j * 16, 16)
                    xb[e, fsl] = jnp.maximum(xb[e, fsl], 0.0) * nv
            return gc

        lax.fori_loop(0, B // 16, grp, 0)

    def body(i, pb, pn, first):
        xb_b, nn_b, dv_b, sn_b, sw_b, sx_b, ss_b = pb
        xb_n, nn_n, dv_n, sn_n, sw_n, sx_n, ss_n = pn
        nn_wait(nn_b, sn_b)
        x_wait(xb_b, sx_b)
        if not first:

            @pl.when(i >= 1)
            def _():
                sc_wait(dv_n, xb_n, ss_n)

        @pl.when(i + 1 < NBLK)
        def _():
            w_issue(i + 1, xb_n, sw_n)
            nn_issue(i + 1, nn_n, sn_n)

        compute(i, xb_b, nn_b, dv_b)
        sc_issue(dv_b, xb_b, ss_b)

        @pl.when(i + 1 < NBLK)
        def _():
            w_wait(xb_n, sw_n)
            x_issue(i + 1, xb_n, sx_n)

    w_issue(0, xb0, sw0)
    nn_issue(0, nn0, sn0)
    w_wait(xb0, sw0)
    x_issue(0, xb0, sx0)
    body(0, bufs[0], bufs[1], first=True)

    def pair(g, carry):
        body(2 * g + 1, bufs[1], bufs[0], first=False)
        body(2 * g + 2, bufs[0], bufs[1], first=False)
        return carry

    lax.fori_loop(0, (NBLK - 1) // 2, pair, 0)
    sc_wait(dv0, xb0, ss0)
    plsc.subcore_barrier()
    pltpu.sync_copy(acc.at[pl.ds(s * NPS, NPS)], out_hbm.at[c, pl.ds(s * NPS, NPS)])


def _tc_node_body(feat_ref, w_ref, root_ref, p0_ref, p1_ref,
                  x_ref, dinv_ref, hinit_ref):
    x = jnp.dot(feat_ref[...], w_ref[...], preferred_element_type=jnp.float32)
    deg = 1.0 + p0_ref[0, :, 0:1] + p1_ref[0, :, 0:1]
    x_ref[...] = x
    dinv_ref[...] = lax.rsqrt(deg)
    hinit_ref[...] = (0.5 * jnp.maximum(x + root_ref[...], 0.0)) * (1.0 / deg)


def _tc_w_body(ef_ref, we_ref, be_ref, w_ref):
    w_ref[...] = (
        jnp.dot(ef_ref[...], we_ref[...], preferred_element_type=jnp.float32)
        + be_ref[...]
    )


def _tc_merge_body(a_ref, b_ref, o_ref):
    o_ref[...] = a_ref[0] + b_ref[0]


def kernel(feat, edge_index, edge_feat, W, We, be, root_emb):
    src = edge_index[0].astype(jnp.int32)
    dst = edge_index[1].astype(jnp.int32)
    dst3 = dst.reshape(NW, NBLK, B)

    feat_p = jnp.pad(feat, ((0, NP - N_NODES), (0, 0)))
    zeros16 = jnp.zeros((NP, 16), jnp.float32)
    ones16 = jnp.ones((B, 16), jnp.float32)
    deg_parts = _sc_degree(dst3, zeros16, ones16)

    rn = 640
    x, dinv, hinit_half = pl.pallas_call(
        _tc_node_body,
        grid=(NP // rn,),
        in_specs=[
            pl.BlockSpec((rn, FEATS), lambda i: (i, 0)),
            pl.BlockSpec((FEATS, FEATS), lambda i: (0, 0)),
            pl.BlockSpec((1, FEATS), lambda i: (0, 0)),
            pl.BlockSpec((1, rn, 16), lambda i: (0, i, 0)),
            pl.BlockSpec((1, rn, 16), lambda i: (1, i, 0)),
        ],
        out_specs=[
            pl.BlockSpec((rn, FEATS), lambda i: (i, 0)),
            pl.BlockSpec((rn, 1), lambda i: (i, 0)),
            pl.BlockSpec((rn, FEATS), lambda i: (i, 0)),
        ],
        out_shape=[
            jax.ShapeDtypeStruct((NP, FEATS), jnp.float32),
            jax.ShapeDtypeStruct((NP, 1), jnp.float32),
            jax.ShapeDtypeStruct((NP, FEATS), jnp.float32),
        ],
    )(feat_p, W, root_emb, deg_parts, deg_parts)
    dinv = dinv.reshape((NP,))

    ef_pad = jnp.pad(edge_feat, ((0, 0), (0, 1)))
    we_pad = jnp.pad(We, ((0, 1), (0, 0)))
    re = 4000
    w = pl.pallas_call(
        _tc_w_body,
        grid=(N_EDGES // re,),
        in_specs=[
            pl.BlockSpec((re, 8), lambda i: (i, 0)),
            pl.BlockSpec((8, FEATS), lambda i: (0, 0)),
            pl.BlockSpec((1, FEATS), lambda i: (0, 0)),
        ],
        out_specs=pl.BlockSpec((re, FEATS), lambda i: (i, 0)),
        out_shape=jax.ShapeDtypeStruct((N_EDGES, FEATS), jnp.float32),
    )(ef_pad, we_pad, be.reshape(1, FEATS))

    parts = _sc_edges(src, dst, x, w, dinv, hinit_half)

    out = pl.pallas_call(
        _tc_merge_body,
        grid=(NP // rn,),
        in_specs=[
            pl.BlockSpec((1, rn, FEATS), lambda i: (0, i, 0)),
            pl.BlockSpec((1, rn, FEATS), lambda i: (1, i, 0)),
        ],
        out_specs=pl.BlockSpec((rn, FEATS), lambda i: (i, 0)),
        out_shape=jax.ShapeDtypeStruct((NP, FEATS), jnp.float32),
    )(parts, parts)
    return out[:N_NODES]


# speedup vs baseline: 12.7183x; 1.1335x over previous
import functools

import jax
import jax.numpy as jnp
from jax import lax
from jax.experimental import pallas as pl
from jax.experimental.pallas import tpu as pltpu
from jax.experimental.pallas import tpu_sc as plsc

N_NODES = 10000
N_EDGES = 320000
FEATS = 128
NP = 10240

NC = 2
NS = 16
NW = NC * NS
EPW = N_EDGES // NW
B = 80
NBLK = EPW // B
NPS = NP // NS

_mesh = plsc.VectorSubcoreMesh(core_axis_name="c", subcore_axis_name="s")


@functools.partial(
    pl.kernel,
    out_type=jax.ShapeDtypeStruct((NC, NP, 16), jnp.float32),
    mesh=_mesh,
    scratch_types=[
        pltpu.VMEM((NBLK, B), jnp.int32),
        pltpu.VMEM((B, 16), jnp.float32),
        pltpu.SemaphoreType.DMA,
        pltpu.SemaphoreType.DMA,
        pltpu.SemaphoreType.DMA,
        pltpu.SemaphoreType.DMA,
        pltpu.SemaphoreType.DMA,
        pltpu.VMEM_SHARED((NP, 16), jnp.float32),
    ],
)
def _sc_degree(dst_hbm, zeros_hbm, ones_hbm, out_hbm,
               id_all, ones_v, s0, s1, s2, s3, s4, acc):
    c = lax.axis_index("c")
    s = lax.axis_index("s")
    wid = s * NC + c
    sems = (s0, s1, s2, s3, s4)
    pltpu.sync_copy(ones_hbm, ones_v)
    pltpu.sync_copy(dst_hbm.at[wid], id_all)
    pltpu.sync_copy(zeros_hbm.at[pl.ds(s * NPS, NPS)], acc.at[pl.ds(s * NPS, NPS)])
    plsc.subcore_barrier()

    def q_body(q, carry):
        for j in range(5):
            r = q * 5 + j

            @pl.when(q > 0)
            def _():
                pltpu.make_async_copy(ones_v, acc.at[id_all.at[0]], sems[j]).wait()

            pltpu.async_copy(ones_v, acc.at[id_all.at[r]], sems[j], add=True)
        return carry

    lax.fori_loop(0, NBLK // 5, q_body, 0)
    for j in range(5):
        pltpu.make_async_copy(ones_v, acc.at[id_all.at[0]], sems[j]).wait()
    plsc.subcore_barrier()
    pltpu.sync_copy(acc.at[pl.ds(s * NPS, NPS)], out_hbm.at[c, pl.ds(s * NPS, NPS)])


@functools.partial(
    pl.kernel,
    out_type=jax.ShapeDtypeStruct((NC, NP, FEATS), jnp.float32),
    mesh=_mesh,
    scratch_types=[
        pltpu.VMEM((B, FEATS), jnp.float32),
        pltpu.VMEM((B, FEATS), jnp.float32),
        pltpu.VMEM((B, FEATS), jnp.float32),
        pltpu.VMEM((2, B), jnp.float32),
        pltpu.VMEM((2, B), jnp.float32),
        pltpu.VMEM((2, B), jnp.float32),
        pltpu.VMEM((2, B), jnp.int32),
        pltpu.VMEM((2, B), jnp.int32),
        pltpu.VMEM((2, B), jnp.int32),
        pltpu.SemaphoreType.DMA,
        pltpu.SemaphoreType.DMA,
        pltpu.SemaphoreType.DMA,
        pltpu.SemaphoreType.DMA,
        pltpu.SemaphoreType.DMA,
        pltpu.SemaphoreType.DMA,
        pltpu.SemaphoreType.DMA,
        pltpu.SemaphoreType.DMA,
        pltpu.SemaphoreType.DMA,
        pltpu.SemaphoreType.DMA,
        pltpu.SemaphoreType.DMA,
        pltpu.SemaphoreType.DMA,
        pltpu.SemaphoreType.DMA,
        pltpu.SemaphoreType.DMA,
        pltpu.SemaphoreType.DMA,
        pltpu.VMEM_SHARED((NP, FEATS), jnp.float32),
    ],
)
def _sc_edges(sd_hbm, x_hbm, w_hbm, dinv_hbm, hinit_hbm, out_hbm,
              xb0, xb1, xb2, nn0, nn1, nn2, sd0, sd1, sd2,
              si0, si1, si2, sn0, sn1, sn2, sw0, sw1, sw2,
              sx0, sx1, sx2, ss0, ss1, ss2, acc):
    c = lax.axis_index("c")
    s = lax.axis_index("s")
    wid = s * NC + c
    ebase = wid * EPW
    rbase = wid * NBLK

    pltpu.sync_copy(hinit_hbm.at[pl.ds(s * NPS, NPS)], acc.at[pl.ds(s * NPS, NPS)])

    bufs = ((xb0, nn0, sd0, si0, sn0, sw0, sx0, ss0),
            (xb1, nn1, sd1, si1, sn1, sw1, sx1, ss1),
            (xb2, nn2, sd2, si2, sn2, sw2, sx2, ss2))

    def idx_issue(i, sd, sem):
        pltpu.async_copy(sd_hbm.at[rbase + i], sd, sem)

    def idx_wait(sd, sem):
        pltpu.make_async_copy(sd_hbm.at[0], sd, sem).wait()

    def w_issue(i, xb, sem):
        pltpu.async_copy(w_hbm.at[pl.ds(ebase + i * B, B)], xb, sem)

    def w_wait(xb, sem):
        pltpu.make_async_copy(w_hbm.at[pl.ds(0, B)], xb, sem).wait()

    def nn_issue(sd, nn, sem):
        pltpu.async_copy(dinv_hbm.at[sd.at[0]], nn.at[0], sem)
        pltpu.async_copy(dinv_hbm.at[sd.at[1]], nn.at[1], sem)

    def nn_wait(nn, sem):
        pltpu.make_async_copy(dinv_hbm.at[pl.ds(0, B)], nn.at[0], sem).wait()
        pltpu.make_async_copy(dinv_hbm.at[pl.ds(0, B)], nn.at[1], sem).wait()

    def x_issue(sd, xb, sem):
        pltpu.async_copy(x_hbm.at[sd.at[0]], xb, sem, add=True)

    def x_wait(xb, sem):
        pltpu.make_async_copy(x_hbm.at[pl.ds(0, B)], xb, sem).wait()

    def sc_issue(sd, xb, sem):
        pltpu.async_copy(xb, acc.at[sd.at[1]], sem, add=True)

    def sc_wait(sd, xb, sem):
        pltpu.make_async_copy(xb, acc.at[sd.at[1]], sem).wait()

    def compute(xb, nn):
        def grp(g, gc):
            sl = pl.ds(g * 16, 16)
            nrm = nn[0, sl] * nn[1, sl]
            for l in range(16):
                nv = nrm[l]
                e = g * 16 + l
                for j in range(FEATS // 16):
                    fsl = pl.ds(j * 16, 16)
                    xb[e, fsl] = jnp.maximum(xb[e, fsl], 0.0) * nv
            return gc

        lax.fori_loop(0, B // 16, grp, 0)

    def body(i, pb, pn2, first):
        xb_b, nn_b, sd_b, si_b, sn_b, sw_b, sx_b, ss_b = pb
        xb_n, nn_n, sd_n, si_n, sn_n, sw_n, sx_n, ss_n = pn2
        nn_wait(nn_b, sn_b)
        x_wait(xb_b, sx_b)
        if not first:
            sc_wait(sd_n, xb_n, ss_n)

        @pl.when(i + 2 < NBLK)
        def _():
            idx_issue(i + 2, sd_n, si_n)
            w_issue(i + 2, xb_n, sw_n)

        compute(xb_b, nn_b)
        sc_issue(sd_b, xb_b, ss_b)

        @pl.when(i + 2 < NBLK)
        def _():
            idx_wait(sd_n, si_n)
            nn_issue(sd_n, nn_n, sn_n)
            w_wait(xb_n, sw_n)
            x_issue(sd_n, xb_n, sx_n)

    for i0 in (0, 1):
        p = bufs[i0]
        idx_issue(i0, p[2], p[3])
        w_issue(i0, p[0], p[5])
        idx_wait(p[2], p[3])
        nn_issue(p[2], p[1], p[4])
        w_wait(p[0], p[5])
        x_issue(p[2], p[0], p[6])
    body(0, bufs[0], bufs[2], first=True)
    body(1, bufs[1], bufs[0], first=False)

    def triple(g, carry):
        i0 = 3 * g + 2
        body(i0, bufs[2], bufs[1], first=False)
        body(i0 + 1, bufs[0], bufs[2], first=False)
        body(i0 + 2, bufs[1], bufs[0], first=False)
        return carry

    lax.fori_loop(0, (NBLK - 2) // 3, triple, 0)
    sc_wait(sd1, xb1, ss1)
    plsc.subcore_barrier()
    pltpu.sync_copy(acc.at[pl.ds(s * NPS, NPS)], out_hbm.at[c, pl.ds(s * NPS, NPS)])


def _tc_node_body(feat_ref, w_ref, root_ref, p0_ref, p1_ref,
                  x_ref, dinv_ref, hinit_ref):
    x = jnp.dot(feat_ref[...], w_ref[...], preferred_element_type=jnp.float32)
    deg = 1.0 + p0_ref[0, :, 0:1] + p1_ref[0, :, 0:1]
    x_ref[...] = x
    dinv_ref[...] = lax.rsqrt(deg)
    hinit_ref[...] = (0.5 * jnp.maximum(x + root_ref[...], 0.0)) * (1.0 / deg)


def _tc_w_body(ef_ref, we_ref, be_ref, w_ref):
    w_ref[...] = (
        jnp.dot(ef_ref[...], we_ref[...], preferred_element_type=jnp.float32)
        + be_ref[...]
    )


def _tc_merge_body(a_ref, b_ref, o_ref):
    o_ref[...] = a_ref[0] + b_ref[0]


def kernel(feat, edge_index, edge_feat, W, We, be, root_emb):
    src = edge_index[0].astype(jnp.int32)
    dst = edge_index[1].astype(jnp.int32)
    dst3 = dst.reshape(NW, NBLK, B)
    sd3 = jnp.stack(
        [src.reshape(NW, NBLK, B), dst3], axis=2
    ).reshape(NW * NBLK, 2, B)

    feat_p = jnp.pad(feat, ((0, NP - N_NODES), (0, 0)))
    zeros16 = jnp.zeros((NP, 16), jnp.float32)
    ones16 = jnp.ones((B, 16), jnp.float32)
    deg_parts = _sc_degree(dst3, zeros16, ones16)

    rn = 640
    x, dinv, hinit_half = pl.pallas_call(
        _tc_node_body,
        grid=(NP // rn,),
        in_specs=[
            pl.BlockSpec((rn, FEATS), lambda i: (i, 0)),
            pl.BlockSpec((FEATS, FEATS), lambda i: (0, 0)),
            pl.BlockSpec((1, FEATS), lambda i: (0, 0)),
            pl.BlockSpec((1, rn, 16), lambda i: (0, i, 0)),
            pl.BlockSpec((1, rn, 16), lambda i: (1, i, 0)),
        ],
        out_specs=[
            pl.BlockSpec((rn, FEATS), lambda i: (i, 0)),
            pl.BlockSpec((rn, 1), lambda i: (i, 0)),
            pl.BlockSpec((rn, FEATS), lambda i: (i, 0)),
        ],
        out_shape=[
            jax.ShapeDtypeStruct((NP, FEATS), jnp.float32),
            jax.ShapeDtypeStruct((NP, 1), jnp.float32),
            jax.ShapeDtypeStruct((NP, FEATS), jnp.float32),
        ],
    )(feat_p, W, root_emb, deg_parts, deg_parts)
    dinv = dinv.reshape((NP,))

    ef_pad = jnp.pad(edge_feat, ((0, 0), (0, 1)))
    we_pad = jnp.pad(We, ((0, 1), (0, 0)))
    re = 4000
    w = pl.pallas_call(
        _tc_w_body,
        grid=(N_EDGES // re,),
        in_specs=[
            pl.BlockSpec((re, 8), lambda i: (i, 0)),
            pl.BlockSpec((8, FEATS), lambda i: (0, 0)),
            pl.BlockSpec((1, FEATS), lambda i: (0, 0)),
        ],
        out_specs=pl.BlockSpec((re, FEATS), lambda i: (i, 0)),
        out_shape=jax.ShapeDtypeStruct((N_EDGES, FEATS), jnp.float32),
    )(ef_pad, we_pad, be.reshape(1, FEATS))

    parts = _sc_edges(sd3, x, w, dinv, hinit_half)

    out = pl.pallas_call(
        _tc_merge_body,
        grid=(NP // rn,),
        in_specs=[
            pl.BlockSpec((1, rn, FEATS), lambda i: (0, i, 0)),
            pl.BlockSpec((1, rn, FEATS), lambda i: (1, i, 0)),
        ],
        out_specs=pl.BlockSpec((rn, FEATS), lambda i: (i, 0)),
        out_shape=jax.ShapeDtypeStruct((NP, FEATS), jnp.float32),
    )(parts, parts)
    return out[:N_NODES]
